# baseline (speedup 1.0000x reference)
import jax
import jax.numpy as jnp
from jax.experimental import pallas as pl
from jax.experimental.pallas import tpu as pltpu
from jax.experimental.pallas import tpu_sc as plsc

_CHUNK = 128
_SPLIT = 2


def _gather_flat(table_p, idx, n, dp):
    mesh = plsc.VectorSubcoreMesh(core_axis_name="c", subcore_axis_name="s")

    @pl.kernel(
        out_type=jax.ShapeDtypeStruct((n, dp), table_p.dtype),
        mesh=mesh,
    )
    def k(table_hbm, idx_hbm, out_hbm):
        def body(i_vmem, o_vmem):
            pltpu.sync_copy(table_hbm.at[i_vmem.at[0]], o_vmem)

        pltpu.emit_pipeline(
            body,
            grid=(n // _CHUNK,),
            in_specs=[pl.BlockSpec((1, _CHUNK), lambda i: (0, i))],
            out_specs=[pl.BlockSpec((_CHUNK, dp), lambda i: (i, 0))],
            core_axis_name=("c", "s"),
            dimension_semantics=(pltpu.PARALLEL,),
        )(idx_hbm, out_hbm)

    return k(table_p, idx)


def kernel(word_ids, table):
    B, H = word_ids.shape
    V, D = table.shape
    Dp = 384
    table_p = jnp.pad(table, ((0, 0), (0, Dp - D)))

    Bs = B // _SPLIT
    parts = []
    for s in range(_SPLIT):
        ids = word_ids[s * Bs:(s + 1) * Bs]
        idx = ids.reshape(1, Bs * H).astype(jnp.int32)
        flat = _gather_flat(table_p, idx, Bs * H, Dp)
        parts.append(flat[:, :D].reshape(Bs, H, D))
    return jnp.concatenate(parts, axis=0)
